# Initial kernel scaffold; baseline (speedup 1.0000x reference)
import jax
import jax.numpy as jnp
from jax.experimental import pallas as pl


def kernel(features, adj, W, b, a1_w, a1_b, a2_w, a2_b):
    raise NotImplementedError("write your pallas kernel here")


# speedup vs baseline: 2.8471x; 2.8471x over previous
import functools

import jax
import jax.numpy as jnp
from jax.experimental import pallas as pl
from jax.experimental.pallas import tpu as pltpu


def _largest_divisor(n, limit, multiple):
    for d in range(min(limit, n), 0, -1):
        if n % d == 0 and d % multiple == 0:
            return d
    return n


def _prep_kernel(f_ref, w_ref, b_ref, a1w_ref, a1b_ref, a2w_ref, a2b_ref,
                 hb_ref, e1p_ref, e1n_ref, e2p_ref, e2n_ref):
    f = f_ref[...]
    w = w_ref[...]
    h = jax.lax.dot_general(f, w, (((1,), (1,)), ((), ())),
                            preferred_element_type=jnp.float32) + b_ref[...]
    hb_ref[...] = h.astype(jnp.bfloat16)
    a1 = jnp.sum(h * a1w_ref[...], axis=1, keepdims=True) + a1b_ref[0, 0]
    a2 = jnp.sum(h * a2w_ref[...], axis=1, keepdims=True) + a2b_ref[0, 0]
    hg = 0.5 * (jnp.max(a1) + jnp.max(a2))
    e1p_ref[...] = jnp.exp(a1 - hg)
    e1n_ref[...] = jnp.exp(0.01 * a1 - hg)
    e2p_ref[...] = jnp.exp(a2 - hg)
    e2n_ref[...] = jnp.exp(0.01 * a2 - hg)


def _gat_kernel(adj_ref, e1p_ref, e1n_ref, e2p_ref, e2n_ref, hb_ref,
                out_ref, acc_ref, den_ref, *, nj, bc):
    j = pl.program_id(1)

    @pl.when(j == 0)
    def _():
        acc_ref[...] = jnp.zeros_like(acc_ref)
        den_ref[...] = jnp.zeros_like(den_ref)

    adjb = adj_ref[...]
    p = jnp.maximum(e1p_ref[...] * e2p_ref[...], e1n_ref[...] * e2n_ref[...])
    p = jnp.where(adjb == 1.0, p, 0.0)
    den_ref[...] += jnp.sum(p, axis=1, keepdims=True)
    hb = hb_ref[pl.ds(j * bc, bc), :]
    acc_ref[...] += jnp.dot(p.astype(jnp.bfloat16), hb,
                            preferred_element_type=jnp.float32)

    @pl.when(j == nj - 1)
    def _():
        d = den_ref[...]
        out_ref[...] = acc_ref[...] / jnp.where(d > 0.0, d, 1.0)


def kernel(features, adj, W, b, a1_w, a1_b, a2_w, a2_b):
    n, d_in = features.shape
    d_out = W.shape[0]
    f32 = jnp.float32

    hb, e1p, e1n, e2p, e2n = pl.pallas_call(
        _prep_kernel,
        out_shape=(
            jax.ShapeDtypeStruct((n, d_out), jnp.bfloat16),
            jax.ShapeDtypeStruct((n, 1), f32),
            jax.ShapeDtypeStruct((n, 1), f32),
            jax.ShapeDtypeStruct((n, 1), f32),
            jax.ShapeDtypeStruct((n, 1), f32),
        ),
    )(features, W, b.reshape(1, d_out), a1_w, a1_b.reshape(1, 1),
      a2_w, a2_b.reshape(1, 1))
    e2p = e2p.reshape(1, n)
    e2n = e2n.reshape(1, n)

    br = _largest_divisor(n, 1024, 8)
    bc = min(2048, ((n + 127) // 128) * 128)
    ni, nj = n // br, pl.cdiv(n, bc)
    n_pad = nj * bc
    if n_pad != n:
        e2p = jnp.pad(e2p, ((0, 0), (0, n_pad - n)))
        e2n = jnp.pad(e2n, ((0, 0), (0, n_pad - n)))
        hb = jnp.pad(hb, ((0, n_pad - n), (0, 0)))

    out = pl.pallas_call(
        functools.partial(_gat_kernel, nj=nj, bc=bc),
        grid=(ni, nj),
        in_specs=[
            pl.BlockSpec((br, bc), lambda i, j: (i, j)),
            pl.BlockSpec((br, 1), lambda i, j: (i, 0)),
            pl.BlockSpec((br, 1), lambda i, j: (i, 0)),
            pl.BlockSpec((1, bc), lambda i, j: (0, j)),
            pl.BlockSpec((1, bc), lambda i, j: (0, j)),
            pl.BlockSpec((n_pad, d_out), lambda i, j: (0, 0)),
        ],
        out_specs=pl.BlockSpec((br, d_out), lambda i, j: (i, 0)),
        out_shape=jax.ShapeDtypeStruct((n, d_out), f32),
        scratch_shapes=[
            pltpu.VMEM((br, d_out), f32),
            pltpu.VMEM((br, 1), f32),
        ],
        compiler_params=pltpu.CompilerParams(
            dimension_semantics=("arbitrary", "arbitrary"),
        ),
    )(adj, e1p, e1n, e2p, e2n, hb)
    return out


# speedup vs baseline: 3.3057x; 1.1611x over previous
import functools

import jax
import jax.numpy as jnp
from jax.experimental import pallas as pl
from jax.experimental.pallas import tpu as pltpu


def _largest_divisor(n, limit, multiple):
    for d in range(min(limit, n), 0, -1):
        if n % d == 0 and d % multiple == 0:
            return d
    return n


def _prep_kernel(f_ref, w_ref, b_ref, a1w_ref, a1b_ref, a2w_ref, a2b_ref,
                 hb_ref, e1p_ref, e1n_ref, e2p_ref, e2n_ref):
    f = f_ref[...]
    w = w_ref[...]
    h = jax.lax.dot_general(f, w, (((1,), (1,)), ((), ())),
                            preferred_element_type=jnp.float32) + b_ref[...]
    hb_ref[...] = h.astype(jnp.bfloat16)
    a1 = jnp.sum(h * a1w_ref[...], axis=1, keepdims=True) + a1b_ref[0, 0]
    a2 = jnp.sum(h * a2w_ref[...], axis=1, keepdims=True) + a2b_ref[0, 0]
    hg = 0.5 * (jnp.max(a1) + jnp.max(a2))
    e1p_ref[...] = jnp.exp(a1 - hg)
    e1n_ref[...] = jnp.exp(0.01 * a1 - hg)
    e2p_ref[...] = jnp.exp(a2 - hg)
    e2n_ref[...] = jnp.exp(0.01 * a2 - hg)


def _gat_kernel(adj_ref, e1p_ref, e1n_ref, e2p_ref, e2n_ref, hb_ref,
                out_ref, acc_ref, *, nj, bc, d_out):
    j = pl.program_id(1)

    @pl.when(j == 0)
    def _():
        acc_ref[...] = jnp.zeros_like(acc_ref)

    adjb = adj_ref[...]
    p = jnp.maximum(e1p_ref[...] * e2p_ref[...], e1n_ref[...] * e2n_ref[...])
    p = jnp.where(adjb == 1.0, p, 0.0)
    hb = hb_ref[pl.ds(j * bc, bc), :]
    acc_ref[...] += jnp.dot(p.astype(jnp.bfloat16), hb,
                            preferred_element_type=jnp.float32)

    @pl.when(j == nj - 1)
    def _():
        d = acc_ref[:, d_out:d_out + 1]
        r = 1.0 / jnp.where(d > 0.0, d, 1.0)
        out_ref[...] = acc_ref[:, :d_out] * r


def kernel(features, adj, W, b, a1_w, a1_b, a2_w, a2_b):
    n, d_in = features.shape
    d_out = W.shape[0]
    f32 = jnp.float32

    hb, e1p, e1n, e2p, e2n = pl.pallas_call(
        _prep_kernel,
        out_shape=(
            jax.ShapeDtypeStruct((n, d_out), jnp.bfloat16),
            jax.ShapeDtypeStruct((n, 1), f32),
            jax.ShapeDtypeStruct((n, 1), f32),
            jax.ShapeDtypeStruct((n, 1), f32),
            jax.ShapeDtypeStruct((n, 1), f32),
        ),
    )(features, W, b.reshape(1, d_out), a1_w, a1_b.reshape(1, 1),
      a2_w, a2_b.reshape(1, 1))
    e2p = e2p.reshape(1, n)
    e2n = e2n.reshape(1, n)

    br = _largest_divisor(n, 1024, 8)
    bc = min(2048, ((n + 127) // 128) * 128)
    ni, nj = n // br, pl.cdiv(n, bc)
    n_pad = nj * bc
    if n_pad != n:
        e2p = jnp.pad(e2p, ((0, 0), (0, n_pad - n)))
        e2n = jnp.pad(e2n, ((0, 0), (0, n_pad - n)))
    d_ext = ((d_out + 1 + 127) // 128) * 128
    hb = jnp.concatenate(
        [hb, jnp.ones((n, 1), jnp.bfloat16),
         jnp.zeros((n, d_ext - d_out - 1), jnp.bfloat16)], axis=1)
    if n_pad != n:
        hb = jnp.pad(hb, ((0, n_pad - n), (0, 0)))

    out = pl.pallas_call(
        functools.partial(_gat_kernel, nj=nj, bc=bc, d_out=d_out),
        grid=(ni, nj),
        in_specs=[
            pl.BlockSpec((br, bc), lambda i, j: (i, j)),
            pl.BlockSpec((br, 1), lambda i, j: (i, 0)),
            pl.BlockSpec((br, 1), lambda i, j: (i, 0)),
            pl.BlockSpec((1, bc), lambda i, j: (0, j)),
            pl.BlockSpec((1, bc), lambda i, j: (0, j)),
            pl.BlockSpec((n_pad, d_ext), lambda i, j: (0, 0)),
        ],
        out_specs=pl.BlockSpec((br, d_out), lambda i, j: (i, 0)),
        out_shape=jax.ShapeDtypeStruct((n, d_out), f32),
        scratch_shapes=[
            pltpu.VMEM((br, d_ext), f32),
        ],
        compiler_params=pltpu.CompilerParams(
            dimension_semantics=("arbitrary", "arbitrary"),
        ),
    )(adj, e1p, e1n, e2p, e2n, hb)
    return out


# speedup vs baseline: 3.4058x; 1.0303x over previous
import functools

import jax
import jax.numpy as jnp
from jax.experimental import pallas as pl
from jax.experimental.pallas import tpu as pltpu


def _largest_divisor(n, limit, multiple):
    for d in range(min(limit, n), 0, -1):
        if n % d == 0 and d % multiple == 0:
            return d
    return n


def _prep_kernel(f_ref, w_ref, b_ref, a1w_ref, a1b_ref, a2w_ref, a2b_ref,
                 hb_ref, e1p_ref, e1n_ref, e2p_ref, e2n_ref):
    f = f_ref[...]
    w = w_ref[...]
    h = jax.lax.dot_general(f, w, (((1,), (1,)), ((), ())),
                            preferred_element_type=jnp.float32) + b_ref[...]
    hb_ref[...] = h.astype(jnp.bfloat16)
    a1 = jnp.sum(h * a1w_ref[...], axis=1, keepdims=True) + a1b_ref[0, 0]
    a2 = jnp.sum(h * a2w_ref[...], axis=1, keepdims=True) + a2b_ref[0, 0]
    hg = 0.5 * (jnp.max(a1) + jnp.max(a2))
    e1p_ref[...] = jnp.exp(a1 - hg)
    e1n_ref[...] = jnp.exp(0.01 * a1 - hg)
    e2p_ref[...] = jnp.exp(a2 - hg)
    e2n_ref[...] = jnp.exp(0.01 * a2 - hg)


def _gat_kernel(*refs, br, d_out, nway):
    adj_refs = refs[:nway]
    e1p_ref, e1n_ref, e2p_ref, e2n_ref, hb_ref, out_ref = refs[nway:]
    bf = jnp.bfloat16
    e2pb = e2p_ref[...].astype(bf)
    e2nb = e2n_ref[...].astype(bf)
    hb = hb_ref[...]
    for k in range(nway):
        adjb = adj_refs[k][...]
        sl = slice(k * br, (k + 1) * br)
        e1pb = e1p_ref[sl, :].astype(bf)
        e1nb = e1n_ref[sl, :].astype(bf)
        p = jnp.maximum(e1pb * e2pb, e1nb * e2nb)
        p = jnp.where(adjb == 1.0, p, bf(0))
        acc = jnp.dot(p, hb, preferred_element_type=jnp.float32)
        d = acc[:, d_out:d_out + 1]
        r = 1.0 / jnp.where(d > 0.0, d, 1.0)
        out_ref[sl, :] = acc[:, :d_out] * r


def kernel(features, adj, W, b, a1_w, a1_b, a2_w, a2_b):
    n, d_in = features.shape
    d_out = W.shape[0]
    f32 = jnp.float32

    hb, e1p, e1n, e2p, e2n = pl.pallas_call(
        _prep_kernel,
        out_shape=(
            jax.ShapeDtypeStruct((n, d_out), jnp.bfloat16),
            jax.ShapeDtypeStruct((n, 1), f32),
            jax.ShapeDtypeStruct((n, 1), f32),
            jax.ShapeDtypeStruct((n, 1), f32),
            jax.ShapeDtypeStruct((n, 1), f32),
        ),
    )(features, W, b.reshape(1, d_out), a1_w, a1_b.reshape(1, 1),
      a2_w, a2_b.reshape(1, 1))
    e2p = e2p.reshape(1, n)
    e2n = e2n.reshape(1, n)

    nway, bt = 1, n
    for nw in (2, 1):
        d = _largest_divisor(n, 256 * nw, 8 * nw)
        if n % d == 0 and d % (8 * nw) == 0:
            nway, bt = nw, d
            break
    br = bt // nway
    ni = n // bt
    n_pad = ((n + 127) // 128) * 128
    if n_pad != n:
        e2p = jnp.pad(e2p, ((0, 0), (0, n_pad - n)))
        e2n = jnp.pad(e2n, ((0, 0), (0, n_pad - n)))
    d_ext = ((d_out + 1 + 127) // 128) * 128
    hb = jnp.concatenate(
        [hb, jnp.ones((n, 1), jnp.bfloat16),
         jnp.zeros((n, d_ext - d_out - 1), jnp.bfloat16)], axis=1)
    if n_pad != n:
        hb = jnp.pad(hb, ((0, n_pad - n), (0, 0)))

    adj_specs = [
        pl.BlockSpec((br, n_pad), lambda i, k=k: (i * nway + k, 0))
        for k in range(nway)
    ]
    out = pl.pallas_call(
        functools.partial(_gat_kernel, br=br, d_out=d_out, nway=nway),
        grid=(ni,),
        in_specs=adj_specs + [
            pl.BlockSpec((bt, 1), lambda i: (i, 0)),
            pl.BlockSpec((bt, 1), lambda i: (i, 0)),
            pl.BlockSpec((1, n_pad), lambda i: (0, 0)),
            pl.BlockSpec((1, n_pad), lambda i: (0, 0)),
            pl.BlockSpec((n_pad, d_ext), lambda i: (0, 0)),
        ],
        out_specs=pl.BlockSpec((bt, d_out), lambda i: (i, 0)),
        out_shape=jax.ShapeDtypeStruct((n, d_out), f32),
        compiler_params=pltpu.CompilerParams(
            dimension_semantics=("arbitrary",),
        ),
    )(*([adj] * nway), e1p, e1n, e2p, e2n, hb)
    return out


# speedup vs baseline: 3.5411x; 1.0397x over previous
import functools

import jax
import jax.numpy as jnp
from jax.experimental import pallas as pl
from jax.experimental.pallas import tpu as pltpu


def _largest_divisor(n, limit, multiple):
    for d in range(min(limit, n), 0, -1):
        if n % d == 0 and d % multiple == 0:
            return d
    return n


def _prep_kernel(f_ref, w_ref, b_ref, a1w_ref, a1b_ref, a2w_ref, a2b_ref,
                 hb_ref, e1p_ref, e1n_ref, e2p_ref, e2n_ref, *, n, d_out):
    bf = jnp.bfloat16
    f = f_ref[...]
    w = w_ref[...]
    h = jax.lax.dot_general(f, w, (((1,), (1,)), ((), ())),
                            preferred_element_type=jnp.float32) + b_ref[...]
    hb_ref[...] = jnp.zeros_like(hb_ref)
    hb_ref[0:n, 0:d_out] = h.astype(bf)
    hb_ref[0:n, d_out:d_out + 1] = jnp.ones((n, 1), bf)
    a1 = jnp.sum(h * a1w_ref[...], axis=1, keepdims=True) + a1b_ref[0, 0]
    a2 = jnp.sum(h * a2w_ref[...], axis=1, keepdims=True) + a2b_ref[0, 0]
    hg = 0.5 * (jnp.max(a1) + jnp.max(a2))
    e1p_ref[...] = jnp.exp(a1 - hg).astype(bf)
    e1n_ref[...] = jnp.exp(0.01 * a1 - hg).astype(bf)
    e2p_ref[...] = jnp.exp(a2 - hg).astype(bf)
    e2n_ref[...] = jnp.exp(0.01 * a2 - hg).astype(bf)


def _gat_kernel(*refs, br, d_out, nway):
    adj_refs = refs[:nway]
    e1p_ref, e1n_ref, e2p_ref, e2n_ref, hb_ref, out_ref = refs[nway:]
    bf = jnp.bfloat16
    e2pb = e2p_ref[...]
    e2nb = e2n_ref[...]
    hb = hb_ref[...]
    for k in range(nway):
        adjb = adj_refs[k][...]
        sl = slice(k * br, (k + 1) * br)
        e1pb = e1p_ref[sl, :]
        e1nb = e1n_ref[sl, :]
        p = jnp.maximum(e1pb * e2pb, e1nb * e2nb)
        p = jnp.where(adjb == 1.0, p, bf(0))
        acc = jnp.dot(p, hb, preferred_element_type=jnp.float32)
        d = acc[:, d_out:d_out + 1]
        r = 1.0 / jnp.where(d > 0.0, d, 1.0)
        out_ref[sl, :] = acc[:, :d_out] * r


def kernel(features, adj, W, b, a1_w, a1_b, a2_w, a2_b):
    n, d_in = features.shape
    d_out = W.shape[0]
    f32 = jnp.float32

    nway, bt = 1, n
    for nw in (2, 1):
        d = _largest_divisor(n, 256 * nw, 8 * nw)
        if n % d == 0 and d % (8 * nw) == 0:
            nway, bt = nw, d
            break
    br = bt // nway
    ni = n // bt
    n_pad = ((n + 127) // 128) * 128
    d_ext = ((d_out + 1 + 127) // 128) * 128

    bf = jnp.bfloat16
    hb, e1p, e1n, e2p, e2n = pl.pallas_call(
        functools.partial(_prep_kernel, n=n, d_out=d_out),
        out_shape=(
            jax.ShapeDtypeStruct((n_pad, d_ext), bf),
            jax.ShapeDtypeStruct((n, 1), bf),
            jax.ShapeDtypeStruct((n, 1), bf),
            jax.ShapeDtypeStruct((n, 1), bf),
            jax.ShapeDtypeStruct((n, 1), bf),
        ),
    )(features, W, b.reshape(1, d_out), a1_w, a1_b.reshape(1, 1),
      a2_w, a2_b.reshape(1, 1))
    e2p = jnp.pad(e2p.reshape(1, n), ((0, 0), (0, n_pad - n)))
    e2n = jnp.pad(e2n.reshape(1, n), ((0, 0), (0, n_pad - n)))

    adj_specs = [
        pl.BlockSpec((br, n_pad), lambda i, k=k: (i * nway + k, 0))
        for k in range(nway)
    ]
    out = pl.pallas_call(
        functools.partial(_gat_kernel, br=br, d_out=d_out, nway=nway),
        grid=(ni,),
        in_specs=adj_specs + [
            pl.BlockSpec((bt, 1), lambda i: (i, 0)),
            pl.BlockSpec((bt, 1), lambda i: (i, 0)),
            pl.BlockSpec((1, n_pad), lambda i: (0, 0)),
            pl.BlockSpec((1, n_pad), lambda i: (0, 0)),
            pl.BlockSpec((n_pad, d_ext), lambda i: (0, 0)),
        ],
        out_specs=pl.BlockSpec((bt, d_out), lambda i: (i, 0)),
        out_shape=jax.ShapeDtypeStruct((n, d_out), f32),
        compiler_params=pltpu.CompilerParams(
            dimension_semantics=("arbitrary",),
        ),
    )(*([adj] * nway), e1p, e1n, e2p, e2n, hb)
    return out


# speedup vs baseline: 4.0629x; 1.1474x over previous
import functools

import jax
import jax.numpy as jnp
from jax.experimental import pallas as pl
from jax.experimental.pallas import tpu as pltpu


def _largest_divisor(n, limit, multiple):
    for d in range(min(limit, n), 0, -1):
        if n % d == 0 and d % multiple == 0:
            return d
    return n


def _gat_kernel(f_ref, w_ref, b_ref, a1w_ref, a1b_ref, a2w_ref, a2b_ref,
                *refs, n, br, d_out, nway):
    adj_refs = refs[:nway]
    out_ref, hbx_ref, e1p_ref, e1n_ref, e2p_ref, e2n_ref = refs[nway:]
    bf = jnp.bfloat16
    i = pl.program_id(0)

    @pl.when(i == 0)
    def _():
        h = jax.lax.dot_general(f_ref[...], w_ref[...],
                                (((1,), (1,)), ((), ())),
                                preferred_element_type=jnp.float32)
        h = h + b_ref[...]
        hbx_ref[...] = jnp.zeros_like(hbx_ref)
        hbx_ref[0:n, 0:d_out] = h.astype(bf)
        hbx_ref[0:n, d_out:d_out + 1] = jnp.ones((n, 1), bf)
        a1 = jnp.sum(h * a1w_ref[...], axis=1, keepdims=True) + a1b_ref[0, 0]
        a2t = jax.lax.dot_general(a2w_ref[...], h, (((1,), (1,)), ((), ())),
                                  preferred_element_type=jnp.float32)
        a2t = a2t + a2b_ref[0, 0]
        hg = 0.5 * (jnp.max(a1) + jnp.max(a2t))
        e1p_ref[...] = jnp.exp(a1 - hg)
        e1n_ref[...] = jnp.exp(0.01 * a1 - hg)
        e2p_ref[...] = jnp.zeros_like(e2p_ref)
        e2n_ref[...] = jnp.zeros_like(e2n_ref)
        e2p_ref[0:1, 0:n] = jnp.exp(a2t - hg).astype(bf)
        e2n_ref[0:1, 0:n] = jnp.exp(0.01 * a2t - hg).astype(bf)

    e2pb = e2p_ref[...]
    e2nb = e2n_ref[...]
    hb = hbx_ref[...]
    for k in range(nway):
        adjb = adj_refs[k][...]
        sl = slice(k * br, (k + 1) * br)
        e1pb = e1p_ref[pl.ds(i * br * nway + k * br, br), :].astype(bf)
        e1nb = e1n_ref[pl.ds(i * br * nway + k * br, br), :].astype(bf)
        p = jnp.maximum(e1pb * e2pb, e1nb * e2nb)
        p = jnp.where(adjb == 1.0, p, bf(0))
        acc = jnp.dot(p, hb, preferred_element_type=jnp.float32)
        d = acc[:, d_out:d_out + 1]
        r = 1.0 / jnp.where(d > 0.0, d, 1.0)
        out_ref[sl, :] = acc[:, :d_out] * r


def kernel(features, adj, W, b, a1_w, a1_b, a2_w, a2_b):
    n, d_in = features.shape
    d_out = W.shape[0]
    f32 = jnp.float32
    bf = jnp.bfloat16

    nway, bt = 1, n
    for nw in (1,):
        d = _largest_divisor(n, 256 * nw, 8 * nw)
        if n % d == 0 and d % (8 * nw) == 0:
            nway, bt = nw, d
            break
    br = bt // nway
    ni = n // bt
    n_pad = ((n + 127) // 128) * 128
    d_ext = ((d_out + 1 + 127) // 128) * 128

    adj_specs = [
        pl.BlockSpec((br, n_pad), lambda i, k=k: (i * nway + k, 0))
        for k in range(nway)
    ]
    out = pl.pallas_call(
        functools.partial(_gat_kernel, n=n, br=br, d_out=d_out, nway=nway),
        grid=(ni,),
        in_specs=[
            pl.BlockSpec((n, d_in), lambda i: (0, 0)),
            pl.BlockSpec((d_out, d_in), lambda i: (0, 0)),
            pl.BlockSpec((1, d_out), lambda i: (0, 0)),
            pl.BlockSpec((1, d_out), lambda i: (0, 0)),
            pl.BlockSpec((1, 1), lambda i: (0, 0)),
            pl.BlockSpec((1, d_out), lambda i: (0, 0)),
            pl.BlockSpec((1, 1), lambda i: (0, 0)),
        ] + adj_specs,
        out_specs=pl.BlockSpec((bt, d_out), lambda i: (i, 0)),
        out_shape=jax.ShapeDtypeStruct((n, d_out), f32),
        scratch_shapes=[
            pltpu.VMEM((n_pad, d_ext), bf),
            pltpu.VMEM((n, 1), f32),
            pltpu.VMEM((n, 1), f32),
            pltpu.VMEM((1, n_pad), bf),
            pltpu.VMEM((1, n_pad), bf),
        ],
        compiler_params=pltpu.CompilerParams(
            dimension_semantics=("arbitrary",),
        ),
    )(features, W, b.reshape(1, d_out), a1_w, a1_b.reshape(1, 1),
      a2_w, a2_b.reshape(1, 1), *([adj] * nway))
    return out
